# Initial kernel scaffold; baseline (speedup 1.0000x reference)
import jax
import jax.numpy as jnp
from jax.experimental import pallas as pl


def kernel(hidden_states, boundary_mask):
    raise NotImplementedError("write your pallas kernel here")


# speedup vs baseline: 1.7279x; 1.7279x over previous
import functools

import jax
import jax.numpy as jnp
from jax import lax
from jax.experimental import pallas as pl
from jax.experimental.pallas import tpu as pltpu
from jax.experimental.pallas import tpu_sc as plsc

B, L, D = 4, 8192, 1024
MAXC = L // 4
LANES = 16
NCHUNK = L // LANES
ROWS_PER_W = (B * MAXC) // 32
GK = 32
NG = ROWS_PER_W // GK
DV = D // LANES


def _body(hs_hbm, mask_hbm, out_hbm, tix_hbm, nc_hbm,
          mask_v, c1_v, tix_v, nc_v, idx_v, ncv_v, gbufA, gbufB,
          sp_tix, sp_nc,
          gsemA, gsemB, wsemA, wsemB):
    cid = lax.axis_index("c")
    sid = lax.axis_index("s")
    iota = lax.iota(jnp.int32, LANES)

    @pl.when(sid < B)
    def _phase_a():
        b = sid
        pltpu.sync_copy(mask_hbm.at[b], mask_v)

        def scan_body(c, carry):
            m = mask_v[pl.ds(c * LANES, LANES)]
            c1 = jnp.cumsum(m) + carry
            c1_v[pl.ds(c * LANES, LANES)] = c1
            pos = iota + c * LANES
            dest = c1 - 1
            valid = (m > 0) & (dest < MAXC)
            plsc.store_scatter(tix_v, [dest], pos, mask=valid)
            return jnp.max(c1)

        total = lax.fori_loop(0, NCHUNK, scan_body, jnp.int32(0))

        @pl.when(total < MAXC)
        def _zeros_pass():
            def zbody(c, carry):
                m = mask_v[pl.ds(c * LANES, LANES)]
                c1 = c1_v[pl.ds(c * LANES, LANES)]
                pos = iota + c * LANES
                dest = total + pos - c1
                valid = (m == 0) & (dest < MAXC)
                plsc.store_scatter(tix_v, [dest], pos, mask=valid)
                return carry
            lax.fori_loop(0, NCHUNK, zbody, jnp.int32(0))

        nc_v[...] = jnp.full((LANES,), total, jnp.int32)
        pltpu.sync_copy(tix_v, sp_tix.at[b])
        pltpu.sync_copy(nc_v, sp_nc.at[b])

        @pl.when(cid == 0)
        def _out():
            pltpu.sync_copy(tix_v, tix_hbm.at[b])
            pltpu.sync_copy(nc_v, nc_hbm.at[b])

    plsc.subcore_barrier()

    wid = cid * 16 + sid
    b = wid // 8
    loff = (wid % 8) * ROWS_PER_W
    row0 = wid * ROWS_PER_W

    pltpu.sync_copy(sp_tix.at[b, pl.ds(loff, ROWS_PER_W)], idx_v)
    pltpu.sync_copy(sp_nc.at[b], ncv_v)
    total = jnp.max(ncv_v[...])
    bias = b * L
    for i in range(ROWS_PER_W // LANES):
        idx_v[pl.ds(i * LANES, LANES)] = idx_v[pl.ds(i * LANES, LANES)] + bias

    gbufs = (gbufA, gbufB)
    gsems = (gsemA, gsemB)
    wsems = (wsemA, wsemB)

    def start_gather(g, nb):
        pltpu.async_copy(hs_hbm.at[idx_v.at[pl.ds(g * GK, GK)]],
                         gbufs[nb], gsems[nb])

    def wait_gather(g, nb):
        pltpu.make_async_copy(hs_hbm.at[idx_v.at[pl.ds(g * GK, GK)]],
                              gbufs[nb], gsems[nb]).wait()

    def start_write(g, nb):
        pltpu.async_copy(gbufs[nb], out_hbm.at[pl.ds(row0 + g * GK, GK)],
                         wsems[nb])

    def wait_write(g, nb):
        pltpu.make_async_copy(gbufs[nb],
                              out_hbm.at[pl.ds(row0 + g * GK, GK)],
                              wsems[nb]).wait()

    start_gather(0, 0)
    for g in range(NG):
        nb = g % 2
        if g + 1 < NG:
            if g + 1 >= 2:
                wait_write(g - 1, (g + 1) % 2)
            start_gather(g + 1, (g + 1) % 2)
        wait_gather(g, nb)

        jbase = loff + g * GK
        @pl.when(jbase + GK - 1 >= total)
        def _mask_block(nb=nb, jbase=jbase):
            buf = gbufs[nb]
            def prow(r, carry):
                f = (jbase + r < total).astype(jnp.float32)
                def pvec(v, c2):
                    x = buf[r, pl.ds(v * LANES, LANES)]
                    buf[r, pl.ds(v * LANES, LANES)] = x * f
                    return c2
                lax.fori_loop(0, DV, pvec, jnp.int32(0))
                return carry
            lax.fori_loop(0, GK, prow, jnp.int32(0))

        start_write(g, nb)
    wait_write(NG - 2, NG % 2)
    wait_write(NG - 1, (NG - 1) % 2)


@jax.jit
def _chunk_kernel(hs_flat, mask_i32):
    mesh = plsc.VectorSubcoreMesh(core_axis_name="c", subcore_axis_name="s")
    kern = pl.kernel(
        _body,
        out_type=(
            jax.ShapeDtypeStruct((B * MAXC, D), jnp.float32),
            jax.ShapeDtypeStruct((B, MAXC), jnp.int32),
            jax.ShapeDtypeStruct((B, LANES), jnp.int32),
        ),
        mesh=mesh,
        compiler_params=pltpu.CompilerParams(needs_layout_passes=False),
        scratch_types=(
            pltpu.VMEM((L,), jnp.int32),
            pltpu.VMEM((L,), jnp.int32),
            pltpu.VMEM((MAXC,), jnp.int32),
            pltpu.VMEM((LANES,), jnp.int32),
            pltpu.VMEM((ROWS_PER_W,), jnp.int32),
            pltpu.VMEM((LANES,), jnp.int32),
            pltpu.VMEM((GK, D), jnp.float32),
            pltpu.VMEM((GK, D), jnp.float32),
            pltpu.VMEM_SHARED((B, MAXC), jnp.int32),
            pltpu.VMEM_SHARED((B, LANES), jnp.int32),
            pltpu.SemaphoreType.DMA,
            pltpu.SemaphoreType.DMA,
            pltpu.SemaphoreType.DMA,
            pltpu.SemaphoreType.DMA,
        ),
    )
    return kern(hs_flat, mask_i32)


def kernel(hidden_states, boundary_mask):
    hs_flat = hidden_states.reshape(B * L, D)
    mask_i32 = boundary_mask.astype(jnp.int32)
    chunked_flat, take_idx, nc = _chunk_kernel(hs_flat, mask_i32)
    chunked = chunked_flat.reshape(B, MAXC, D)
    num_chunks = jnp.minimum(nc[:, 0], MAXC)
    pad_mask = jnp.arange(MAXC, dtype=jnp.int32)[None, :] < num_chunks[:, None]
    return (chunked, pad_mask, take_idx)


# speedup vs baseline: 1.8045x; 1.0444x over previous
import functools

import jax
import jax.numpy as jnp
from jax import lax
from jax.experimental import pallas as pl
from jax.experimental.pallas import tpu as pltpu
from jax.experimental.pallas import tpu_sc as plsc

B, L, D = 4, 8192, 1024
MAXC = L // 4
LANES = 16
NCHUNK = L // LANES
ROWS_PER_W = (B * MAXC) // 32
GK = 32
NG = ROWS_PER_W // GK
DV = D // LANES


def _body(hs_hbm, mask_hbm, out_hbm, tix_hbm, nc_hbm,
          mask_v, c1_v, tix_v, nc_v, idx_v, ncv_v, gbufA, gbufB, gbufC,
          sp_tix, sp_nc,
          gsemA, gsemB, gsemC, wsemA, wsemB, wsemC):
    cid = lax.axis_index("c")
    sid = lax.axis_index("s")
    iota = lax.iota(jnp.int32, LANES)

    @pl.when(sid < B)
    def _phase_a():
        b = sid
        pltpu.sync_copy(mask_hbm.at[b], mask_v)

        def scan_body(c, carry_vec):
            m = mask_v[pl.ds(c * LANES, LANES)]
            mb = m > 0
            c1 = jnp.cumsum(m) + carry_vec
            c1_v[pl.ds(c * LANES, LANES)] = c1
            pos = iota + c * LANES
            dest = c1 - 1
            valid = mb & (dest < MAXC)
            plsc.store_scatter(tix_v, [dest], pos, mask=valid)
            return carry_vec + plsc.all_reduce_population_count(mb)

        total_vec = lax.fori_loop(0, NCHUNK, scan_body,
                                  jnp.zeros((LANES,), jnp.int32))
        total = jnp.max(total_vec)

        @pl.when(total < MAXC)
        def _zeros_pass():
            def zbody(c, carry):
                m = mask_v[pl.ds(c * LANES, LANES)]
                c1 = c1_v[pl.ds(c * LANES, LANES)]
                pos = iota + c * LANES
                dest = total + pos - c1
                valid = (m == 0) & (dest < MAXC)
                plsc.store_scatter(tix_v, [dest], pos, mask=valid)
                return carry
            lax.fori_loop(0, NCHUNK, zbody, jnp.int32(0))

        nc_v[...] = jnp.full((LANES,), total, jnp.int32)
        pltpu.sync_copy(tix_v, sp_tix.at[b])
        pltpu.sync_copy(nc_v, sp_nc.at[b])

        @pl.when(cid == 0)
        def _out():
            pltpu.sync_copy(tix_v, tix_hbm.at[b])
            pltpu.sync_copy(nc_v, nc_hbm.at[b])

    plsc.subcore_barrier()

    wid = cid * 16 + sid
    b = wid // 8
    loff = (wid % 8) * ROWS_PER_W
    row0 = wid * ROWS_PER_W

    pltpu.sync_copy(sp_tix.at[b, pl.ds(loff, ROWS_PER_W)], idx_v)
    pltpu.sync_copy(sp_nc.at[b], ncv_v)
    total = jnp.max(ncv_v[...])
    bias = b * L
    for i in range(ROWS_PER_W // LANES):
        idx_v[pl.ds(i * LANES, LANES)] = idx_v[pl.ds(i * LANES, LANES)] + bias

    gbufs = (gbufA, gbufB, gbufC)
    gsems = (gsemA, gsemB, gsemC)
    wsems = (wsemA, wsemB, wsemC)
    NBUF = 3

    def start_gather(g, nb):
        pltpu.async_copy(hs_hbm.at[idx_v.at[pl.ds(g * GK, GK)]],
                         gbufs[nb], gsems[nb])

    def wait_gather(g, nb):
        pltpu.make_async_copy(hs_hbm.at[idx_v.at[pl.ds(g * GK, GK)]],
                              gbufs[nb], gsems[nb]).wait()

    def start_write(g, nb):
        pltpu.async_copy(gbufs[nb], out_hbm.at[pl.ds(row0 + g * GK, GK)],
                         wsems[nb])

    def wait_write(g, nb):
        pltpu.make_async_copy(gbufs[nb],
                              out_hbm.at[pl.ds(row0 + g * GK, GK)],
                              wsems[nb]).wait()

    start_gather(0, 0)
    start_gather(1, 1)
    for g in range(NG):
        nb = g % NBUF
        nxt = g + 2
        if nxt < NG:
            nbx = nxt % NBUF
            if nxt >= NBUF:
                wait_write(nxt - NBUF, nbx)
            start_gather(nxt, nbx)
        wait_gather(g, nb)

        jbase = loff + g * GK
        @pl.when(jbase + GK - 1 >= total)
        def _mask_block(nb=nb, jbase=jbase):
            buf = gbufs[nb]
            def prow(r, carry):
                f = (jbase + r < total).astype(jnp.float32)
                def pvec(v, c2):
                    x = buf[r, pl.ds(v * LANES, LANES)]
                    buf[r, pl.ds(v * LANES, LANES)] = x * f
                    return c2
                lax.fori_loop(0, DV, pvec, jnp.int32(0))
                return carry
            lax.fori_loop(0, GK, prow, jnp.int32(0))

        start_write(g, nb)
    for g in range(max(0, NG - 3), NG):
        wait_write(g, g % NBUF)


@jax.jit
def _chunk_kernel(hs_flat, mask_i32):
    mesh = plsc.VectorSubcoreMesh(core_axis_name="c", subcore_axis_name="s")
    kern = pl.kernel(
        _body,
        out_type=(
            jax.ShapeDtypeStruct((B * MAXC, D), jnp.float32),
            jax.ShapeDtypeStruct((B, MAXC), jnp.int32),
            jax.ShapeDtypeStruct((B, LANES), jnp.int32),
        ),
        mesh=mesh,
        compiler_params=pltpu.CompilerParams(needs_layout_passes=False),
        scratch_types=(
            pltpu.VMEM((L,), jnp.int32),
            pltpu.VMEM((L,), jnp.int32),
            pltpu.VMEM((MAXC,), jnp.int32),
            pltpu.VMEM((LANES,), jnp.int32),
            pltpu.VMEM((ROWS_PER_W,), jnp.int32),
            pltpu.VMEM((LANES,), jnp.int32),
            pltpu.VMEM((GK, D), jnp.float32),
            pltpu.VMEM((GK, D), jnp.float32),
            pltpu.VMEM((GK, D), jnp.float32),
            pltpu.VMEM_SHARED((B, MAXC), jnp.int32),
            pltpu.VMEM_SHARED((B, LANES), jnp.int32),
            pltpu.SemaphoreType.DMA,
            pltpu.SemaphoreType.DMA,
            pltpu.SemaphoreType.DMA,
            pltpu.SemaphoreType.DMA,
            pltpu.SemaphoreType.DMA,
            pltpu.SemaphoreType.DMA,
        ),
    )
    return kern(hs_flat, mask_i32)


def kernel(hidden_states, boundary_mask):
    hs_flat = hidden_states.reshape(B * L, D)
    mask_i32 = boundary_mask.astype(jnp.int32)
    chunked_flat, take_idx, nc = _chunk_kernel(hs_flat, mask_i32)
    chunked = chunked_flat.reshape(B, MAXC, D)
    num_chunks = jnp.minimum(nc[:, 0], MAXC)
    pad_mask = jnp.arange(MAXC, dtype=jnp.int32)[None, :] < num_chunks[:, None]
    return (chunked, pad_mask, take_idx)
